# baseline (speedup 1.0000x reference)
import jax
import jax.numpy as jnp
from jax.experimental import pallas as pl

_N = 1_000_000
_OBS_VAL = 1000.0


_C = 131_072


def _tc_indicator(xt):
    def body(x_ref, o_ref):
        e = x_ref[0:1, :]
        o = x_ref[1:2, :]
        m = (jnp.abs(e) <= 3.0) & (jnp.abs(o) <= 1.5)
        o_ref[...] = jnp.where(m, jnp.float32(_OBS_VAL), jnp.float32(0.0))

    return pl.pallas_call(
        body,
        grid=(-(-_N // _C),),
        in_specs=[pl.BlockSpec((2, _C), lambda i: (0, i))],
        out_specs=pl.BlockSpec((1, _C), lambda i: (0, i)),
        out_shape=jax.ShapeDtypeStruct((1, _N), jnp.float32),
    )(xt)


def kernel(x):
    out = _tc_indicator(x.T)
    return out.reshape(_N, 1)
